# baseline (speedup 1.0000x reference)
import functools

import jax
import jax.numpy as jnp
from jax import lax
from jax.experimental import pallas as pl
from jax.experimental.pallas import tpu as pltpu
from jax.experimental.pallas import tpu_sc as plsc

EMBED = 64
LANES = 16
NC, NS = 2, 16
NW = NC * NS
BATCH = 16
CTX = 2048
TOTAL = BATCH * CTX
BPW = TOTAL // NW
CHUNK = 128
NCHUNK = BPW // CHUNK
SCALE = 8.0
VOCAB = 1000000
VROWS = 2000
TCGRID = VOCAB // VROWS


def _tc_compact(tab):
    rows = lax.broadcasted_iota(jnp.int32, (VROWS // 2, VROWS), 0)
    cols = lax.broadcasted_iota(jnp.int32, (VROWS // 2, VROWS), 1)
    sel_e = (cols == 2 * rows).astype(jnp.float32)
    sel_o = (cols == 2 * rows + 1).astype(jnp.float32)

    def body(se_ref, so_ref, in_ref, out_ref):
        x = in_ref[...] * SCALE
        out_ref[:, 0:EMBED] = jax.lax.dot_general(
            se_ref[...], x, (((1,), (0,)), ((), ())),
            preferred_element_type=jnp.float32)
        out_ref[:, EMBED:2 * EMBED] = jax.lax.dot_general(
            so_ref[...], x, (((1,), (0,)), ((), ())),
            preferred_element_type=jnp.float32)

    return pl.pallas_call(
        body,
        grid=(TCGRID,),
        in_specs=[
            pl.BlockSpec((VROWS // 2, VROWS), lambda i: (0, 0)),
            pl.BlockSpec((VROWS // 2, VROWS), lambda i: (0, 0)),
            pl.BlockSpec((VROWS, EMBED), lambda i: (i, 0)),
        ],
        out_specs=pl.BlockSpec((VROWS // 2, 2 * EMBED), lambda i: (i, 0)),
        out_shape=jax.ShapeDtypeStruct((VOCAB // 2, 2 * EMBED), jnp.float32),
    )(sel_e, sel_o, tab)


def _sc_embed(X, tabc, pos):
    mesh = plsc.VectorSubcoreMesh(core_axis_name="c", subcore_axis_name="s")

    @functools.partial(
        pl.kernel,
        out_type=jax.ShapeDtypeStruct((BATCH, CTX, EMBED), jnp.float32),
        mesh=mesh,
        scratch_types=[
            pltpu.VMEM((BPW,), jnp.int32),
            pltpu.VMEM((NCHUNK, CHUNK), jnp.int32),
            pltpu.VMEM((BPW,), jnp.int32),
            pltpu.VMEM((CHUNK, 2 * EMBED), jnp.float32),
            pltpu.VMEM((CHUNK, 2 * EMBED), jnp.float32),
            pltpu.VMEM((CHUNK, EMBED), jnp.float32),
            pltpu.VMEM((CHUNK, EMBED), jnp.float32),
            pltpu.VMEM((CHUNK, EMBED), jnp.float32),
            pltpu.VMEM((CHUNK, EMBED), jnp.float32),
            pltpu.SemaphoreType.DMA,
            pltpu.SemaphoreType.DMA,
            pltpu.SemaphoreType.DMA,
            pltpu.SemaphoreType.DMA,
            pltpu.SemaphoreType.DMA,
            pltpu.SemaphoreType.DMA,
        ],
        compiler_params=pltpu.CompilerParams(
            needs_layout_passes=False, use_tc_tiling_on_sc=True),
    )
    def k(x_hbm, tab_hbm, pos_hbm, out_hbm, idx_v, q_v, cb_v,
          pr_a, pr_b, pos_a, pos_b, ot_a, ot_b,
          gsem_a, gsem_b, psem_a, psem_b, wsem_a, wsem_b):
        wid = lax.axis_index("s") * NC + lax.axis_index("c")
        b = wid // 2
        t0 = (wid % 2) * BPW

        prs = (pr_a, pr_b)
        poss = (pos_a, pos_b)
        ots = (ot_a, ot_b)
        gsems = (gsem_a, gsem_b)
        psems = (psem_a, psem_b)
        wsems = (wsem_a, wsem_b)

        pltpu.sync_copy(x_hbm.at[b, pl.ds(t0, BPW)], idx_v)

        def prep(rr, _):
            for cc in range(CHUNK // LANES):
                sl = pl.ds(rr * CHUNK + cc * LANES, LANES)
                v = idx_v[sl]
                q_v[rr, pl.ds(cc * LANES, LANES)] = (
                    lax.shift_right_arithmetic(v, 1))
                cb_v[sl] = lax.shift_left(jnp.bitwise_and(v, 1), 6)
            return 0

        lax.fori_loop(0, NCHUNK, prep, 0)

        lane = lax.iota(jnp.int32, LANES)

        def fire_chunk(j, par):
            pltpu.async_copy(tab_hbm.at[q_v.at[j]], prs[par], gsems[par])
            pltpu.async_copy(
                pos_hbm.at[0, pl.ds(t0 + j * CHUNK, CHUNK)],
                poss[par], psems[par])

        def drain_chunk(par):
            pltpu.make_async_copy(
                tab_hbm.at[pl.ds(0, CHUNK)], prs[par], gsems[par]).wait()
            pltpu.make_async_copy(
                pos_hbm.at[0, pl.ds(0, CHUNK)], poss[par], psems[par]).wait()

        fire_chunk(0, 0)

        wb = [None] * NCHUNK
        for j in range(NCHUNK):
            par = j % 2
            if j + 1 < NCHUNK:
                if j >= 1 and wb[j - 1] is not None:
                    wb[j - 1].wait()
                    wb[j - 1] = None
                fire_chunk(j + 1, 1 - par)
            drain_chunk(par)

            pr, posb, ot = prs[par], poss[par], ots[par]
            jfull = jnp.full((LANES,), j, dtype=jnp.int32)

            def row_body(r, _):
                rfull = jnp.full((LANES,), r, dtype=jnp.int32)
                cb16 = plsc.load_gather(
                    cb_v, [jnp.full((LANES,), j * CHUNK, jnp.int32) + rfull])
                for c in range(EMBED // LANES):
                    cols = cb16 + (c * LANES + lane)
                    src = plsc.load_gather(pr, [rfull, cols])
                    dst = (r, pl.ds(c * LANES, LANES))
                    ot[dst] = src + posb[dst]
                return 0

            lax.fori_loop(0, CHUNK, row_body, 0)
            wb[j] = pltpu.async_copy(
                ot, out_hbm.at[b, pl.ds(t0 + j * CHUNK, CHUNK)], wsems[par])

        for h in wb:
            if h is not None:
                h.wait()

    return k(X, tabc, pos)


def kernel(X, tok_table, pos_embedding):
    tabc = _tc_compact(tok_table)
    return _sc_embed(X, tabc, pos_embedding)


# speedup vs baseline: 1.8412x; 1.8412x over previous
import functools

import jax
import jax.numpy as jnp
from jax import lax
from jax.experimental import pallas as pl
from jax.experimental.pallas import tpu as pltpu
from jax.experimental.pallas import tpu_sc as plsc

EMBED = 64
LANES = 16
NC, NS = 2, 16
NW = NC * NS
BATCH = 16
CTX = 2048
TOTAL = BATCH * CTX
BPW = TOTAL // NW
CHUNK = 128
NCHUNK = BPW // CHUNK
SCALE = 8.0
VOCAB = 1000000
G = 160
NG = VOCAB // G
MAXT = -(-(NG // NW + 1) // 2) * 2


def _sc_compact(tab):
    mesh = plsc.VectorSubcoreMesh(core_axis_name="c", subcore_axis_name="s")

    @functools.partial(
        pl.kernel,
        out_type=jax.ShapeDtypeStruct((VOCAB // 2, 2 * EMBED), jnp.float32),
        mesh=mesh,
        scratch_types=[
            pltpu.VMEM((G, EMBED), jnp.float32),
            pltpu.VMEM((G, EMBED), jnp.float32),
            pltpu.VMEM((G // 2, 2 * EMBED), jnp.float32),
            pltpu.VMEM((G // 2, 2 * EMBED), jnp.float32),
            pltpu.SemaphoreType.DMA,
            pltpu.SemaphoreType.DMA,
            pltpu.SemaphoreType.DMA,
            pltpu.SemaphoreType.DMA,
        ],
        compiler_params=pltpu.CompilerParams(
            needs_layout_passes=False, use_tc_tiling_on_sc=True),
    )
    def k(tab_hbm, outc_hbm, in_a, in_b, out_a, out_b,
          rsem_a, rsem_b, wsem_a, wsem_b):
        wid = lax.axis_index("s") * NC + lax.axis_index("c")
        n_w = NG // NW + jnp.where(wid < NG % NW, 1, 0)

        ins = (in_a, in_b)
        outs = (out_a, out_b)
        rsems = (rsem_a, rsem_b)
        wsems = (wsem_a, wsem_b)

        def chunk_of(t):
            return wid + NW * t

        def fire_read(t, par):
            pltpu.async_copy(
                tab_hbm.at[pl.ds(chunk_of(t) * G, G)], ins[par], rsems[par])

        def wait_read(par):
            pltpu.make_async_copy(
                tab_hbm.at[pl.ds(0, G)], ins[par], rsems[par]).wait()

        def wait_write(par):
            pltpu.make_async_copy(
                outs[par], outc_hbm.at[pl.ds(0, G // 2)], wsems[par]).wait()

        def remap_and_write(t, par):
            inb, outb = ins[par], outs[par]

            def row_body(r, _):
                for h in range(2):
                    for c in range(EMBED // LANES):
                        src = (2 * r + h, pl.ds(c * LANES, LANES))
                        dst = (r, pl.ds(h * EMBED + c * LANES, LANES))
                        outb[dst] = inb[src]
                return 0

            lax.fori_loop(0, G // 2, row_body, 0)
            pltpu.async_copy(
                outb, outc_hbm.at[pl.ds(chunk_of(t) * (G // 2), G // 2)],
                wsems[par])

        fire_read(0, 0)

        def pair_body(i, _):
            ta = 2 * i
            tb = 2 * i + 1

            @pl.when(tb < n_w)
            def _():
                fire_read(tb, 1)

            @pl.when(jnp.logical_and(ta >= 2, ta - 2 < n_w))
            def _():
                wait_write(0)

            @pl.when(ta < n_w)
            def _():
                wait_read(0)
                remap_and_write(ta, 0)

            @pl.when(ta + 2 < n_w)
            def _():
                fire_read(ta + 2, 0)

            @pl.when(jnp.logical_and(tb >= 2, tb - 2 < n_w))
            def _():
                wait_write(1)

            @pl.when(tb < n_w)
            def _():
                wait_read(1)
                remap_and_write(tb, 1)

            return 0

        lax.fori_loop(0, MAXT // 2, pair_body, 0)
        wait_write(0)

        @pl.when(n_w % 2 == 0)
        def _():
            wait_write(1)

    return k(tab)


def _sc_embed(X, tabc, pos):
    mesh = plsc.VectorSubcoreMesh(core_axis_name="c", subcore_axis_name="s")

    @functools.partial(
        pl.kernel,
        out_type=jax.ShapeDtypeStruct((BATCH, CTX, EMBED), jnp.float32),
        mesh=mesh,
        scratch_types=[
            pltpu.VMEM((BPW,), jnp.int32),
            pltpu.VMEM((NCHUNK, CHUNK), jnp.int32),
            pltpu.VMEM((BPW,), jnp.int32),
            pltpu.VMEM((CHUNK, 2 * EMBED), jnp.float32),
            pltpu.VMEM((CHUNK, 2 * EMBED), jnp.float32),
            pltpu.VMEM((CHUNK, EMBED), jnp.float32),
            pltpu.VMEM((CHUNK, EMBED), jnp.float32),
            pltpu.VMEM((CHUNK, EMBED), jnp.float32),
            pltpu.VMEM((CHUNK, EMBED), jnp.float32),
            pltpu.SemaphoreType.DMA,
            pltpu.SemaphoreType.DMA,
            pltpu.SemaphoreType.DMA,
            pltpu.SemaphoreType.DMA,
            pltpu.SemaphoreType.DMA,
            pltpu.SemaphoreType.DMA,
        ],
        compiler_params=pltpu.CompilerParams(
            needs_layout_passes=False, use_tc_tiling_on_sc=True),
    )
    def k(x_hbm, tab_hbm, pos_hbm, out_hbm, idx_v, q_v, cb_v,
          pr_a, pr_b, pos_a, pos_b, ot_a, ot_b,
          gsem_a, gsem_b, psem_a, psem_b, wsem_a, wsem_b):
        wid = lax.axis_index("s") * NC + lax.axis_index("c")
        b = wid // 2
        t0 = (wid % 2) * BPW

        prs = (pr_a, pr_b)
        poss = (pos_a, pos_b)
        ots = (ot_a, ot_b)
        gsems = (gsem_a, gsem_b)
        psems = (psem_a, psem_b)
        wsems = (wsem_a, wsem_b)

        pltpu.sync_copy(x_hbm.at[b, pl.ds(t0, BPW)], idx_v)

        def prep(rr, _):
            for cc in range(CHUNK // LANES):
                sl = pl.ds(rr * CHUNK + cc * LANES, LANES)
                v = idx_v[sl]
                q_v[rr, pl.ds(cc * LANES, LANES)] = (
                    lax.shift_right_arithmetic(v, 1))
                cb_v[sl] = lax.shift_left(jnp.bitwise_and(v, 1), 6)
            return 0

        lax.fori_loop(0, NCHUNK, prep, 0)

        lane = lax.iota(jnp.int32, LANES)

        def fire_chunk(j, par):
            pltpu.async_copy(tab_hbm.at[q_v.at[j]], prs[par], gsems[par])
            pltpu.async_copy(
                pos_hbm.at[0, pl.ds(t0 + j * CHUNK, CHUNK)],
                poss[par], psems[par])

        def drain_chunk(par):
            pltpu.make_async_copy(
                tab_hbm.at[pl.ds(0, CHUNK)], prs[par], gsems[par]).wait()
            pltpu.make_async_copy(
                pos_hbm.at[0, pl.ds(0, CHUNK)], poss[par], psems[par]).wait()

        fire_chunk(0, 0)

        wb = [None] * NCHUNK
        for j in range(NCHUNK):
            par = j % 2
            if j + 1 < NCHUNK:
                if j >= 1 and wb[j - 1] is not None:
                    wb[j - 1].wait()
                    wb[j - 1] = None
                fire_chunk(j + 1, 1 - par)
            drain_chunk(par)

            pr, posb, ot = prs[par], poss[par], ots[par]

            def row_body(r, _):
                rfull = jnp.full((LANES,), r, dtype=jnp.int32)
                cb16 = plsc.load_gather(
                    cb_v, [jnp.full((LANES,), j * CHUNK, jnp.int32) + rfull])
                for c in range(EMBED // LANES):
                    cols = cb16 + (c * LANES + lane)
                    src = plsc.load_gather(pr, [rfull, cols])
                    dst = (r, pl.ds(c * LANES, LANES))
                    ot[dst] = src * SCALE + posb[dst]
                return 0

            lax.fori_loop(0, CHUNK, row_body, 0)
            wb[j] = pltpu.async_copy(
                ot, out_hbm.at[b, pl.ds(t0 + j * CHUNK, CHUNK)], wsems[par])

        for h in wb:
            if h is not None:
                h.wait()

    return k(X, tabc, pos)


def kernel(X, tok_table, pos_embedding):
    tabc = _sc_compact(tok_table)
    return _sc_embed(X, tabc, pos_embedding)


# speedup vs baseline: 1.9624x; 1.0658x over previous
import functools

import jax
import jax.numpy as jnp
from jax import lax
from jax.experimental import pallas as pl
from jax.experimental.pallas import tpu as pltpu
from jax.experimental.pallas import tpu_sc as plsc

EMBED = 64
LANES = 16
NC, NS = 2, 16
NW = NC * NS
BATCH = 16
CTX = 2048
TOTAL = BATCH * CTX
BPW = TOTAL // NW
CHUNK = 128
NCHUNK = BPW // CHUNK
SCALE = 8.0
HALF_V = 500000


def _sc_embed(X, tabc, pos):
    mesh = plsc.VectorSubcoreMesh(core_axis_name="c", subcore_axis_name="s")

    @functools.partial(
        pl.kernel,
        out_type=jax.ShapeDtypeStruct((BATCH, CTX, EMBED), jnp.float32),
        mesh=mesh,
        scratch_types=[
            pltpu.VMEM((BPW,), jnp.int32),
            pltpu.VMEM((NCHUNK, CHUNK), jnp.int32),
            pltpu.VMEM((BPW,), jnp.int32),
            pltpu.VMEM((CHUNK, 2 * EMBED), jnp.float32),
            pltpu.VMEM((CHUNK, 2 * EMBED), jnp.float32),
            pltpu.VMEM((CHUNK, EMBED), jnp.float32),
            pltpu.VMEM((CHUNK, EMBED), jnp.float32),
            pltpu.VMEM((CHUNK, EMBED), jnp.float32),
            pltpu.VMEM((CHUNK, EMBED), jnp.float32),
            pltpu.SemaphoreType.DMA,
            pltpu.SemaphoreType.DMA,
            pltpu.SemaphoreType.DMA,
            pltpu.SemaphoreType.DMA,
            pltpu.SemaphoreType.DMA,
            pltpu.SemaphoreType.DMA,
        ],
        compiler_params=pltpu.CompilerParams(
            needs_layout_passes=False, use_tc_tiling_on_sc=True),
    )
    def k(x_hbm, tab_hbm, pos_hbm, out_hbm, idx_v, q_v, cb_v,
          pr_a, pr_b, pos_a, pos_b, ot_a, ot_b,
          gsem_a, gsem_b, psem_a, psem_b, wsem_a, wsem_b):
        wid = lax.axis_index("s") * NC + lax.axis_index("c")
        b = wid // 2
        t0 = (wid % 2) * BPW

        prs = (pr_a, pr_b)
        poss = (pos_a, pos_b)
        ots = (ot_a, ot_b)
        gsems = (gsem_a, gsem_b)
        psems = (psem_a, psem_b)
        wsems = (wsem_a, wsem_b)

        pltpu.sync_copy(x_hbm.at[b, pl.ds(t0, BPW)], idx_v)

        def prep(rr, _):
            for cc in range(CHUNK // LANES):
                sl = pl.ds(rr * CHUNK + cc * LANES, LANES)
                v = idx_v[sl]
                hi = (v >= HALF_V).astype(jnp.int32)
                q_v[rr, pl.ds(cc * LANES, LANES)] = v - hi * HALF_V
                cb_v[sl] = lax.shift_left(hi, 6)
            return 0

        lax.fori_loop(0, NCHUNK, prep, 0)

        lane = lax.iota(jnp.int32, LANES)

        def fire_chunk(j, par):
            pltpu.async_copy(tab_hbm.at[q_v.at[j]], prs[par], gsems[par])
            pltpu.async_copy(
                pos_hbm.at[0, pl.ds(t0 + j * CHUNK, CHUNK)],
                poss[par], psems[par])

        def drain_chunk(par):
            pltpu.make_async_copy(
                tab_hbm.at[pl.ds(0, CHUNK)], prs[par], gsems[par]).wait()
            pltpu.make_async_copy(
                pos_hbm.at[0, pl.ds(0, CHUNK)], poss[par], psems[par]).wait()

        fire_chunk(0, 0)

        wb = [None] * NCHUNK
        for j in range(NCHUNK):
            par = j % 2
            if j + 1 < NCHUNK:
                if j >= 1 and wb[j - 1] is not None:
                    wb[j - 1].wait()
                    wb[j - 1] = None
                fire_chunk(j + 1, 1 - par)
            drain_chunk(par)

            pr, posb, ot = prs[par], poss[par], ots[par]

            def row_body(r, _):
                rfull = jnp.full((LANES,), r, dtype=jnp.int32)
                cb16 = plsc.load_gather(
                    cb_v, [jnp.full((LANES,), j * CHUNK, jnp.int32) + rfull])
                for c in range(EMBED // LANES):
                    cols = cb16 + (c * LANES + lane)
                    src = plsc.load_gather(pr, [rfull, cols])
                    dst = (r, pl.ds(c * LANES, LANES))
                    ot[dst] = src * SCALE + posb[dst]
                return 0

            lax.fori_loop(0, CHUNK, row_body, 0)
            wb[j] = pltpu.async_copy(
                ot, out_hbm.at[b, pl.ds(t0 + j * CHUNK, CHUNK)], wsems[par])

        for h in wb:
            if h is not None:
                h.wait()

    return k(X, tabc, pos)


def kernel(X, tok_table, pos_embedding):
    tabc = jnp.concatenate(
        [tok_table[:HALF_V], tok_table[HALF_V:]], axis=1)
    return _sc_embed(X, tabc, pos_embedding)


# speedup vs baseline: 4.0345x; 2.0559x over previous
import functools

import jax
import jax.numpy as jnp
from jax import lax
from jax.experimental import pallas as pl
from jax.experimental.pallas import tpu as pltpu
from jax.experimental.pallas import tpu_sc as plsc

EMBED = 64
LANES = 16
NC, NS = 2, 16
NW = NC * NS
BATCH = 16
CTX = 2048
TOTAL = BATCH * CTX
BPW = TOTAL // NW
CHUNK = 128
NCHUNK = BPW // CHUNK
SCALE = 8.0


def _sc_embed(X, tab, pos):
    mesh = plsc.VectorSubcoreMesh(core_axis_name="c", subcore_axis_name="s")

    @functools.partial(
        pl.kernel,
        out_type=jax.ShapeDtypeStruct((BATCH, CTX, EMBED), jnp.float32),
        mesh=mesh,
        scratch_types=[
            pltpu.VMEM((BPW,), jnp.int32),
            pltpu.VMEM((CHUNK, EMBED), jnp.float32),
            pltpu.VMEM((CHUNK, EMBED), jnp.float32),
            pltpu.VMEM((CHUNK, EMBED), jnp.float32),
            pltpu.VMEM((CHUNK, EMBED), jnp.float32),
            pltpu.SemaphoreType.DMA,
            pltpu.SemaphoreType.DMA,
            pltpu.SemaphoreType.DMA,
            pltpu.SemaphoreType.DMA,
            pltpu.SemaphoreType.DMA,
            pltpu.SemaphoreType.DMA,
        ],
        compiler_params=pltpu.CompilerParams(
            needs_layout_passes=False, use_tc_tiling_on_sc=True),
    )
    def k(x_hbm, tab_hbm, pos_hbm, out_hbm, idx_v, buf_a, buf_b,
          pos_a, pos_b, gsem_a, gsem_b, wsem_a, wsem_b, psem_a, psem_b):
        wid = lax.axis_index("s") * NC + lax.axis_index("c")
        b = wid // 2
        t0 = (wid % 2) * BPW

        bufs = (buf_a, buf_b)
        poss = (pos_a, pos_b)
        gsems = (gsem_a, gsem_b)
        wsems = (wsem_a, wsem_b)
        psems = (psem_a, psem_b)

        pltpu.sync_copy(x_hbm.at[b, pl.ds(t0, BPW)], idx_v)

        def fire_chunk(j, par):
            buf, gsem = bufs[par], gsems[par]
            pltpu.async_copy(
                pos_hbm.at[0, pl.ds(t0 + j * CHUNK, CHUNK)],
                poss[par], psems[par])

            def fire_group(i, _):
                v16 = idx_v[pl.ds(j * CHUNK + i * LANES, LANES)]
                for r in range(LANES):
                    pltpu.async_copy(
                        tab_hbm.at[pl.ds(v16[r], 1)],
                        buf.at[pl.ds(i * LANES + r, 1)], gsem)
                return 0

            lax.fori_loop(0, CHUNK // LANES, fire_group, 0)

        def drain_chunk(par):
            pltpu.make_async_copy(
                tab_hbm.at[pl.ds(0, CHUNK)], bufs[par], gsems[par]).wait()
            pltpu.make_async_copy(
                pos_hbm.at[0, pl.ds(0, CHUNK)], poss[par], psems[par]).wait()

        fire_chunk(0, 0)

        wb = [None] * NCHUNK
        for j in range(NCHUNK):
            par = j % 2
            if j + 1 < NCHUNK:
                if j >= 1 and wb[j - 1] is not None:
                    wb[j - 1].wait()
                    wb[j - 1] = None
                fire_chunk(j + 1, 1 - par)
            drain_chunk(par)

            buf, posb = bufs[par], poss[par]

            def row_body(r, _):
                for c in range(EMBED // LANES):
                    sl = (r, pl.ds(c * LANES, LANES))
                    buf[sl] = buf[sl] * SCALE + posb[sl]
                return 0

            lax.fori_loop(0, CHUNK, row_body, 0)
            wb[j] = pltpu.async_copy(
                buf, out_hbm.at[b, pl.ds(t0 + j * CHUNK, CHUNK)], wsems[par])

        for h in wb:
            if h is not None:
                h.wait()

    return k(X, tab, pos)


def kernel(X, tok_table, pos_embedding):
    return _sc_embed(X, tok_table, pos_embedding)
